# Initial kernel scaffold; baseline (speedup 1.0000x reference)
import jax
import jax.numpy as jnp
from jax.experimental import pallas as pl


def kernel(embeddings, token_ids):
    raise NotImplementedError("write your pallas kernel here")


# speedup vs baseline: 2.9645x; 2.9645x over previous
import functools

import jax
import jax.numpy as jnp
from jax import lax
from jax.experimental import pallas as pl
from jax.experimental.pallas import tpu as pltpu
from jax.experimental.pallas import tpu_sc as plsc

_NUM_CORES = 2
_NUM_SUBCORES = 16
_NUM_WORKERS = _NUM_CORES * _NUM_SUBCORES

_CHUNK = 128


def _make_gather(num_rows: int, embed_dim: int, n_chunks: int):
  mesh = plsc.VectorSubcoreMesh(core_axis_name="c", subcore_axis_name="s")
  rows_per_worker = n_chunks * _CHUNK

  @functools.partial(
      pl.kernel,
      mesh=mesh,
      out_type=jax.ShapeDtypeStruct((num_rows, embed_dim), jnp.float32),
      scratch_types=[
          pltpu.VMEM((n_chunks, _CHUNK), jnp.int32),
          pltpu.VMEM((_CHUNK, embed_dim), jnp.float32),
          pltpu.SemaphoreType.DMA,
      ],
  )
  def gather_kernel(table_hbm, idx_hbm, out_hbm, idx_v, rows_v, sem):
    wid = lax.axis_index("s") * _NUM_CORES + lax.axis_index("c")
    base = wid * rows_per_worker
    pltpu.sync_copy(idx_hbm.at[wid], idx_v)

    @pl.loop(0, n_chunks)
    def _(j):
      pltpu.async_copy(table_hbm.at[idx_v.at[j]], rows_v, sem).wait()
      pltpu.sync_copy(rows_v, out_hbm.at[pl.ds(base + j * _CHUNK, _CHUNK)])

  return gather_kernel


def kernel(embeddings, token_ids):
  batch, hist = token_ids.shape
  vocab, embed_dim = embeddings.shape
  num_rows = batch * hist
  assert num_rows % (_NUM_WORKERS * _CHUNK) == 0
  n_chunks = num_rows // (_NUM_WORKERS * _CHUNK)

  idx = token_ids.reshape(_NUM_WORKERS, n_chunks, _CHUNK).astype(jnp.int32)
  out = _make_gather(num_rows, embed_dim, n_chunks)(embeddings, idx)
  return out.reshape(batch, hist, embed_dim)


# speedup vs baseline: 3.3453x; 1.1284x over previous
import functools

import jax
import jax.numpy as jnp
from jax import lax
from jax.experimental import pallas as pl
from jax.experimental.pallas import tpu as pltpu
from jax.experimental.pallas import tpu_sc as plsc

_NUM_CORES = 2
_NUM_SUBCORES = 16
_NUM_WORKERS = _NUM_CORES * _NUM_SUBCORES

_CHUNK = 128


_NBUF = 7
_AHEAD = 3


def _make_gather(num_rows: int, embed_dim: int, n_chunks: int):
  mesh = plsc.VectorSubcoreMesh(core_axis_name="c", subcore_axis_name="s")
  rows_per_worker = n_chunks * _CHUNK

  @functools.partial(
      pl.kernel,
      mesh=mesh,
      out_type=jax.ShapeDtypeStruct((num_rows, embed_dim), jnp.float32),
      scratch_types=[
          pltpu.VMEM((n_chunks, _CHUNK), jnp.int32),
          pltpu.VMEM((_NBUF, _CHUNK, embed_dim), jnp.float32),
          pltpu.SemaphoreType.DMA((_NBUF,)),
          pltpu.SemaphoreType.DMA((_NBUF,)),
      ],
  )
  def gather_kernel(table_hbm, idx_hbm, out_hbm, idx_v, rows_v, gsem, ssem):
    wid = lax.axis_index("s") * _NUM_CORES + lax.axis_index("c")
    base = wid * rows_per_worker
    pltpu.sync_copy(idx_hbm.at[wid], idx_v)

    for g in range(_AHEAD):
      pltpu.async_copy(table_hbm.at[idx_v.at[g]], rows_v.at[g], gsem.at[g])

    @pl.loop(0, n_chunks)
    def _(j):
      slot = lax.rem(j, _NBUF)
      g = j + _AHEAD

      @pl.when(g < n_chunks)
      def _():
        gslot = lax.rem(g, _NBUF)

        @pl.when(j >= _NBUF - _AHEAD)
        def _():
          pltpu.make_async_copy(
              rows_v.at[gslot], out_hbm.at[pl.ds(base, _CHUNK)], ssem.at[gslot]
          ).wait()

        pltpu.async_copy(table_hbm.at[idx_v.at[g]], rows_v.at[gslot],
                         gsem.at[gslot])

      pltpu.make_async_copy(
          table_hbm.at[idx_v.at[j]], rows_v.at[slot], gsem.at[slot]
      ).wait()
      pltpu.async_copy(
          rows_v.at[slot], out_hbm.at[pl.ds(base + j * _CHUNK, _CHUNK)],
          ssem.at[slot],
      )

    for b in range(_NBUF):
      pltpu.make_async_copy(
          rows_v.at[b], out_hbm.at[pl.ds(base, _CHUNK)], ssem.at[b]
      ).wait()

  return gather_kernel


def kernel(embeddings, token_ids):
  batch, hist = token_ids.shape
  vocab, embed_dim = embeddings.shape
  num_rows = batch * hist
  assert num_rows % (_NUM_WORKERS * _CHUNK) == 0
  n_chunks = num_rows // (_NUM_WORKERS * _CHUNK)

  idx = token_ids.reshape(_NUM_WORKERS, n_chunks, _CHUNK).astype(jnp.int32)
  out = _make_gather(num_rows, embed_dim, n_chunks)(embeddings, idx)
  return out.reshape(batch, hist, embed_dim)


# speedup vs baseline: 5.9620x; 1.7822x over previous
import functools

import jax
import jax.numpy as jnp
from jax import lax
from jax.experimental import pallas as pl
from jax.experimental.pallas import tpu as pltpu
from jax.experimental.pallas import tpu_sc as plsc

_NUM_CORES = 2
_NUM_SUBCORES = 16
_NUM_WORKERS = _NUM_CORES * _NUM_SUBCORES

_NBUF = 8
_AHEAD = 4


def _make_gather(batch: int, hist: int, embed_dim: int):
  mesh = plsc.VectorSubcoreMesh(core_axis_name="c", subcore_axis_name="s")
  rows_per_worker = batch // _NUM_WORKERS

  @functools.partial(
      pl.kernel,
      mesh=mesh,
      out_type=jax.ShapeDtypeStruct((batch, hist, embed_dim), jnp.float32),
      scratch_types=[
          pltpu.VMEM((rows_per_worker, hist), jnp.int32),
          pltpu.VMEM((_NBUF, hist, embed_dim), jnp.float32),
          pltpu.SemaphoreType.DMA((_NBUF,)),
          pltpu.SemaphoreType.DMA((_NBUF,)),
      ],
  )
  def gather_kernel(table_hbm, idx_hbm, out_hbm, idx_v, rows_v, gsem, ssem):
    wid = lax.axis_index("s") * _NUM_CORES + lax.axis_index("c")
    base = wid * rows_per_worker
    pltpu.sync_copy(idx_hbm.at[pl.ds(base, rows_per_worker)], idx_v)

    for g in range(_AHEAD):
      pltpu.async_copy(table_hbm.at[idx_v.at[g]], rows_v.at[g], gsem.at[g])

    @pl.loop(0, rows_per_worker)
    def _(j):
      slot = lax.rem(j, _NBUF)
      g = j + _AHEAD

      @pl.when(g < rows_per_worker)
      def _():
        gslot = lax.rem(g, _NBUF)

        @pl.when(j >= _NBUF - _AHEAD)
        def _():
          pltpu.make_async_copy(
              rows_v.at[gslot], out_hbm.at[base], ssem.at[gslot]
          ).wait()

        pltpu.async_copy(
            table_hbm.at[idx_v.at[g]], rows_v.at[gslot], gsem.at[gslot]
        )

      pltpu.make_async_copy(
          table_hbm.at[idx_v.at[j]], rows_v.at[slot], gsem.at[slot]
      ).wait()
      pltpu.async_copy(rows_v.at[slot], out_hbm.at[base + j], ssem.at[slot])

    for b in range(_NBUF):
      pltpu.make_async_copy(
          rows_v.at[b], out_hbm.at[base], ssem.at[b]
      ).wait()

  return gather_kernel


def kernel(embeddings, token_ids):
  batch, hist = token_ids.shape
  vocab, embed_dim = embeddings.shape
  assert batch % _NUM_WORKERS == 0
  idx = token_ids.astype(jnp.int32)
  return _make_gather(batch, hist, embed_dim)(embeddings, idx)


# speedup vs baseline: 5.9706x; 1.0015x over previous
import functools

import jax
import jax.numpy as jnp
from jax import lax
from jax.experimental import pallas as pl
from jax.experimental.pallas import tpu as pltpu
from jax.experimental.pallas import tpu_sc as plsc

_NUM_CORES = 2
_NUM_SUBCORES = 16
_NUM_WORKERS = _NUM_CORES * _NUM_SUBCORES

_NBUF = 8
_AHEAD = 4


def _make_gather(batch: int, hist: int, embed_dim: int):
  mesh = plsc.VectorSubcoreMesh(core_axis_name="c", subcore_axis_name="s")
  rows_per_worker = batch // _NUM_WORKERS

  @functools.partial(
      pl.kernel,
      mesh=mesh,
      compiler_params=pltpu.CompilerParams(use_tc_tiling_on_sc=True),
      out_type=jax.ShapeDtypeStruct((batch, hist, embed_dim), jnp.float32),
      scratch_types=[
          pltpu.VMEM((rows_per_worker, hist), jnp.int32),
          pltpu.VMEM((_NBUF, hist, embed_dim), jnp.float32),
          pltpu.SemaphoreType.DMA((_NBUF,)),
          pltpu.SemaphoreType.DMA((_NBUF,)),
      ],
  )
  def gather_kernel(table_hbm, idx_hbm, out_hbm, idx_v, rows_v, gsem, ssem):
    wid = lax.axis_index("s") * _NUM_CORES + lax.axis_index("c")
    base = wid * rows_per_worker
    pltpu.sync_copy(idx_hbm.at[pl.ds(base, rows_per_worker)], idx_v)

    for g in range(_AHEAD):
      pltpu.async_copy(table_hbm.at[idx_v.at[g]], rows_v.at[g], gsem.at[g])

    @pl.loop(0, rows_per_worker)
    def _(j):
      slot = lax.rem(j, _NBUF)
      g = j + _AHEAD

      @pl.when(g < rows_per_worker)
      def _():
        gslot = lax.rem(g, _NBUF)

        @pl.when(j >= _NBUF - _AHEAD)
        def _():
          pltpu.make_async_copy(
              rows_v.at[gslot], out_hbm.at[base], ssem.at[gslot]
          ).wait()

        pltpu.async_copy(
            table_hbm.at[idx_v.at[g]], rows_v.at[gslot], gsem.at[gslot]
        )

      pltpu.make_async_copy(
          table_hbm.at[idx_v.at[j]], rows_v.at[slot], gsem.at[slot]
      ).wait()
      pltpu.async_copy(rows_v.at[slot], out_hbm.at[base + j], ssem.at[slot])

    for b in range(_NBUF):
      pltpu.make_async_copy(
          rows_v.at[b], out_hbm.at[base], ssem.at[b]
      ).wait()

  return gather_kernel


def kernel(embeddings, token_ids):
  batch, hist = token_ids.shape
  vocab, embed_dim = embeddings.shape
  assert batch % _NUM_WORKERS == 0
  idx = token_ids.astype(jnp.int32)
  return _make_gather(batch, hist, embed_dim)(embeddings, idx)


# speedup vs baseline: 10.5555x; 1.7679x over previous
import functools

import jax
import jax.numpy as jnp
from jax import lax
from jax.experimental import pallas as pl
from jax.experimental.pallas import tpu as pltpu
from jax.experimental.pallas import tpu_sc as plsc

_NUM_CORES = 2
_NUM_SUBCORES = 16
_NUM_WORKERS = _NUM_CORES * _NUM_SUBCORES

_CHUNK = 128

_NBUF = 6
_AHEAD = 3


def _make_gather(num_rows: int, embed_dim: int, n_chunks: int):
  mesh = plsc.VectorSubcoreMesh(core_axis_name="c", subcore_axis_name="s")
  rows_per_worker = n_chunks * _CHUNK

  @functools.partial(
      pl.kernel,
      mesh=mesh,
      out_type=jax.ShapeDtypeStruct((num_rows, embed_dim), jnp.float32),
      scratch_types=[
          pltpu.VMEM((rows_per_worker,), jnp.int32),
          pltpu.VMEM((_NBUF, _CHUNK, embed_dim), jnp.float32),
          pltpu.SemaphoreType.DMA((_NBUF,)),
          pltpu.SemaphoreType.DMA((_NBUF,)),
      ],
  )
  def gather_kernel(table_hbm, idx_hbm, out_hbm, idx_v, rows_v, gsem, ssem):
    wid = lax.axis_index("s") * _NUM_CORES + lax.axis_index("c")
    base = wid * rows_per_worker
    pltpu.sync_copy(idx_hbm.at[pl.ds(base, rows_per_worker)], idx_v)

    for g in range(_AHEAD):
      pltpu.async_copy(
          table_hbm.at[idx_v.at[pl.ds(g * _CHUNK, _CHUNK)]],
          rows_v.at[g], gsem.at[g],
      )

    @pl.loop(0, n_chunks)
    def _(j):
      slot = lax.rem(j, _NBUF)
      g = j + _AHEAD

      @pl.when(g < n_chunks)
      def _():
        gslot = lax.rem(g, _NBUF)

        @pl.when(j >= _NBUF - _AHEAD)
        def _():
          pltpu.make_async_copy(
              rows_v.at[gslot], out_hbm.at[pl.ds(base, _CHUNK)],
              ssem.at[gslot],
          ).wait()

        pltpu.async_copy(
            table_hbm.at[idx_v.at[pl.ds(g * _CHUNK, _CHUNK)]],
            rows_v.at[gslot], gsem.at[gslot],
        )

      pltpu.make_async_copy(
          table_hbm.at[idx_v.at[pl.ds(j * _CHUNK, _CHUNK)]],
          rows_v.at[slot], gsem.at[slot],
      ).wait()
      pltpu.async_copy(
          rows_v.at[slot],
          out_hbm.at[pl.ds(base + j * _CHUNK, _CHUNK)],
          ssem.at[slot],
      )

    for b in range(_NBUF):
      pltpu.make_async_copy(
          rows_v.at[b], out_hbm.at[pl.ds(base, _CHUNK)], ssem.at[b]
      ).wait()

  return gather_kernel


def kernel(embeddings, token_ids):
  batch, hist = token_ids.shape
  vocab, embed_dim = embeddings.shape
  num_rows = batch * hist
  assert num_rows % (_NUM_WORKERS * _CHUNK) == 0
  n_chunks = num_rows // (_NUM_WORKERS * _CHUNK)

  idx = jnp.transpose(token_ids).astype(jnp.int32).reshape(num_rows)
  out = _make_gather(num_rows, embed_dim, n_chunks)(embeddings, idx)
  return jnp.transpose(out.reshape(hist, batch, embed_dim), (1, 0, 2))


# speedup vs baseline: 10.5758x; 1.0019x over previous
import functools

import jax
import jax.numpy as jnp
from jax import lax
from jax.experimental import pallas as pl
from jax.experimental.pallas import tpu as pltpu
from jax.experimental.pallas import tpu_sc as plsc

_NUM_CORES = 2
_NUM_SUBCORES = 16
_NUM_WORKERS = _NUM_CORES * _NUM_SUBCORES

_CHUNK = 128

_NBUF = 7
_AHEAD = 4


def _make_gather(num_rows: int, embed_dim: int, n_chunks: int):
  mesh = plsc.VectorSubcoreMesh(core_axis_name="c", subcore_axis_name="s")
  rows_per_worker = n_chunks * _CHUNK

  @functools.partial(
      pl.kernel,
      mesh=mesh,
      out_type=jax.ShapeDtypeStruct((num_rows, embed_dim), jnp.float32),
      scratch_types=[
          pltpu.VMEM((rows_per_worker,), jnp.int32),
          pltpu.VMEM((_NBUF, _CHUNK, embed_dim), jnp.float32),
          pltpu.SemaphoreType.DMA((_NBUF,)),
          pltpu.SemaphoreType.DMA((_NBUF,)),
      ],
  )
  def gather_kernel(table_hbm, idx_hbm, out_hbm, idx_v, rows_v, gsem, ssem):
    wid = lax.axis_index("s") * _NUM_CORES + lax.axis_index("c")
    base = wid * rows_per_worker
    pltpu.sync_copy(idx_hbm.at[pl.ds(base, rows_per_worker)], idx_v)

    for g in range(_AHEAD):
      pltpu.async_copy(
          table_hbm.at[idx_v.at[pl.ds(g * _CHUNK, _CHUNK)]],
          rows_v.at[g], gsem.at[g],
      )

    @pl.loop(0, n_chunks)
    def _(j):
      slot = lax.rem(j, _NBUF)
      g = j + _AHEAD

      @pl.when(g < n_chunks)
      def _():
        gslot = lax.rem(g, _NBUF)

        @pl.when(j >= _NBUF - _AHEAD)
        def _():
          pltpu.make_async_copy(
              rows_v.at[gslot], out_hbm.at[pl.ds(base, _CHUNK)],
              ssem.at[gslot],
          ).wait()

        pltpu.async_copy(
            table_hbm.at[idx_v.at[pl.ds(g * _CHUNK, _CHUNK)]],
            rows_v.at[gslot], gsem.at[gslot],
        )

      pltpu.make_async_copy(
          table_hbm.at[idx_v.at[pl.ds(j * _CHUNK, _CHUNK)]],
          rows_v.at[slot], gsem.at[slot],
      ).wait()
      pltpu.async_copy(
          rows_v.at[slot],
          out_hbm.at[pl.ds(base + j * _CHUNK, _CHUNK)],
          ssem.at[slot],
      )

    for b in range(_NBUF):
      pltpu.make_async_copy(
          rows_v.at[b], out_hbm.at[pl.ds(base, _CHUNK)], ssem.at[b]
      ).wait()

  return gather_kernel


def kernel(embeddings, token_ids):
  batch, hist = token_ids.shape
  vocab, embed_dim = embeddings.shape
  num_rows = batch * hist
  assert num_rows % (_NUM_WORKERS * _CHUNK) == 0
  n_chunks = num_rows // (_NUM_WORKERS * _CHUNK)

  idx = jnp.transpose(token_ids).astype(jnp.int32).reshape(num_rows)
  out = _make_gather(num_rows, embed_dim, n_chunks)(embeddings, idx)
  return jnp.transpose(out.reshape(hist, batch, embed_dim), (1, 0, 2))


# speedup vs baseline: 10.5808x; 1.0005x over previous
import functools

import jax
import jax.numpy as jnp
from jax import lax
from jax.experimental import pallas as pl
from jax.experimental.pallas import tpu as pltpu
from jax.experimental.pallas import tpu_sc as plsc

_NUM_CORES = 2
_NUM_SUBCORES = 16
_NUM_WORKERS = _NUM_CORES * _NUM_SUBCORES

_CHUNK = 128

_NBUF = 7
_AHEAD = 3


def _make_gather(num_rows: int, embed_dim: int, n_chunks: int):
  mesh = plsc.VectorSubcoreMesh(core_axis_name="c", subcore_axis_name="s")
  rows_per_worker = n_chunks * _CHUNK

  @functools.partial(
      pl.kernel,
      mesh=mesh,
      out_type=jax.ShapeDtypeStruct((num_rows, embed_dim), jnp.float32),
      scratch_types=[
          pltpu.VMEM((rows_per_worker,), jnp.int32),
          pltpu.VMEM((_NBUF, _CHUNK, embed_dim), jnp.float32),
          pltpu.SemaphoreType.DMA((_NBUF,)),
          pltpu.SemaphoreType.DMA((_NBUF,)),
      ],
  )
  def gather_kernel(table_hbm, idx_hbm, out_hbm, idx_v, rows_v, gsem, ssem):
    wid = lax.axis_index("s") * _NUM_CORES + lax.axis_index("c")
    base = wid * rows_per_worker
    pltpu.sync_copy(idx_hbm.at[pl.ds(base, rows_per_worker)], idx_v)

    for g in range(_AHEAD):
      pltpu.async_copy(
          table_hbm.at[idx_v.at[pl.ds(g * _CHUNK, _CHUNK)]],
          rows_v.at[g], gsem.at[g],
      )

    @pl.loop(0, n_chunks)
    def _(j):
      slot = lax.rem(j, _NBUF)
      g = j + _AHEAD

      @pl.when(g < n_chunks)
      def _():
        gslot = lax.rem(g, _NBUF)

        @pl.when(j >= _NBUF - _AHEAD)
        def _():
          pltpu.make_async_copy(
              rows_v.at[gslot], out_hbm.at[pl.ds(base, _CHUNK)],
              ssem.at[gslot],
          ).wait()

        pltpu.async_copy(
            table_hbm.at[idx_v.at[pl.ds(g * _CHUNK, _CHUNK)]],
            rows_v.at[gslot], gsem.at[gslot],
        )

      pltpu.make_async_copy(
          table_hbm.at[idx_v.at[pl.ds(j * _CHUNK, _CHUNK)]],
          rows_v.at[slot], gsem.at[slot],
      ).wait()
      pltpu.async_copy(
          rows_v.at[slot],
          out_hbm.at[pl.ds(base + j * _CHUNK, _CHUNK)],
          ssem.at[slot],
      )

    for b in range(_NBUF):
      pltpu.make_async_copy(
          rows_v.at[b], out_hbm.at[pl.ds(base, _CHUNK)], ssem.at[b]
      ).wait()

  return gather_kernel


def kernel(embeddings, token_ids):
  batch, hist = token_ids.shape
  vocab, embed_dim = embeddings.shape
  num_rows = batch * hist
  assert num_rows % (_NUM_WORKERS * _CHUNK) == 0
  n_chunks = num_rows // (_NUM_WORKERS * _CHUNK)

  idx = jnp.transpose(token_ids).astype(jnp.int32).reshape(num_rows)
  out = _make_gather(num_rows, embed_dim, n_chunks)(embeddings, idx)
  return jnp.transpose(out.reshape(hist, batch, embed_dim), (1, 0, 2))
